# Initial kernel scaffold; baseline (speedup 1.0000x reference)
import jax
import jax.numpy as jnp
from jax.experimental import pallas as pl


def kernel(word_h, topic_h, ww_src, ww_dst, ww_w, wt_src, wt_dst, wt_w, wd_src, wd_dst, wd_w, td_src, td_dst, td_w, tt_src, tt_dst, tt_w, W_ww, b_ww, W_wt, b_wt, W_wd, b_wd, W_td, b_td, W_tt, b_tt):
    raise NotImplementedError("write your pallas kernel here")


# speedup vs baseline: 1.2050x; 1.2050x over previous
import functools

import jax
import jax.numpy as jnp
from jax import lax
from jax.experimental import pallas as pl
from jax.experimental.pallas import tpu as pltpu
from jax.experimental.pallas import tpu_sc as plsc

NW, NT, ND = 50000, 10000, 10000
D = 128
R_WW = 12544
R16_WW = R_WW + 128
RT_WW = R16_WW // 16
N16_P2 = 10112
RT_P2 = N16_P2 // 16

_mesh = plsc.VectorSubcoreMesh(core_axis_name="c", subcore_axis_name="s")
_sc_params = pltpu.CompilerParams(needs_layout_passes=False)


def _ceil_to(x, m):
    return (x + m - 1) // m * m


def _pad_edges(src, dst, w, n_dst, ch):
    e = src.shape[0]
    ep = _ceil_to(e, 16 * ch)
    pad = ep - e
    src = jnp.pad(src, (0, pad))
    dst = jnp.pad(dst, (0, pad), constant_values=n_dst)
    w = jnp.pad(w, (0, pad))
    return src, dst, w, ep


def _cw_body(Www, bww, Wwt, bwt, Wwd, bwd, Wtd, btd, Wtt, btt,
             A_o, aG_o, bc_o, B_o, bB_o):
    dn = (((1,), (0,)), ((), ()))
    dt = (((1,), (1,)), ((), ()))
    f32 = jnp.float32
    Wc = lax.dot_general(Wwd[...], Wwt[...], dn, preferred_element_type=f32)
    A_o[...] = lax.dot_general(Wc, Www[...], dn, preferred_element_type=f32)
    aG_o[...] = lax.dot_general(bww[...], Wc, dt, preferred_element_type=f32)
    bc_o[...] = lax.dot_general(bwt[...], Wwd[...], dt,
                                preferred_element_type=f32) + bwd[...]
    B_o[...] = lax.dot_general(Wtt[...], Wtd[...], dn, preferred_element_type=f32)
    bB_o[...] = lax.dot_general(btd[...], Wtt[...], dt,
                                preferred_element_type=f32) + btt[...]


def _combine_weights(W_ww, b_ww, W_wt, b_wt, W_wd, b_wd, W_td, b_td, W_tt, b_tt):
    m = jax.ShapeDtypeStruct((D, D), jnp.float32)
    v = jax.ShapeDtypeStruct((1, D), jnp.float32)
    return pl.pallas_call(
        _cw_body,
        out_shape=[m, v, v, m, v],
    )(W_ww, b_ww.reshape(1, D), W_wt, b_wt.reshape(1, D),
      W_wd, b_wd.reshape(1, D), W_td, b_td.reshape(1, D),
      W_tt, b_tt.reshape(1, D))


def _mm_body(x_ref, w_ref, b_ref, o_ref):
    dt = (((1,), (1,)), ((), ()))
    o_ref[...] = lax.dot_general(
        x_ref[...], w_ref[...], dt, preferred_element_type=jnp.float32
    ) + b_ref[...]


def _matmul(x, W, b):
    n = x.shape[0]
    bm = 512
    grid = (n + bm - 1) // bm
    return pl.pallas_call(
        _mm_body,
        grid=(grid,),
        in_specs=[
            pl.BlockSpec((bm, D), lambda i: (i, 0)),
            pl.BlockSpec((D, D), lambda i: (0, 0)),
            pl.BlockSpec((1, D), lambda i: (0, 0)),
        ],
        out_specs=pl.BlockSpec((bm, D), lambda i: (i, 0)),
        out_shape=jax.ShapeDtypeStruct((n, D), jnp.float32),
    )(x, W, b)


def _comb1_body(s_ref, c_ref, bc_ref, o_ref):
    cnt = jnp.sum(c_ref[0], axis=0)
    cnt = jnp.maximum(cnt, 1.0)
    o_ref[...] = s_ref[0] / cnt[:, None] + bc_ref[...]


def _combine_ww(sum4, cnt4, bc):
    bm = 256
    nb = R_WW // bm
    return pl.pallas_call(
        _comb1_body,
        grid=(4, nb),
        in_specs=[
            pl.BlockSpec((1, bm, D), lambda r, i: (r, i, 0)),
            pl.BlockSpec((1, 16, bm), lambda r, i: (r, 0, i)),
            pl.BlockSpec((1, D), lambda r, i: (0, 0)),
        ],
        out_specs=pl.BlockSpec((bm, D), lambda r, i: (r * nb + i, 0)),
        out_shape=jax.ShapeDtypeStruct((4 * R_WW, D), jnp.float32),
    )(sum4, cnt4, bc)


def _comb2_body(sa_ref, ca_ref, sb_ref, cb_ref, o_ref):
    ca = jnp.maximum(jnp.sum(ca_ref[...], axis=1), 1.0)
    cb = jnp.maximum(jnp.sum(cb_ref[...], axis=1), 1.0)
    o_ref[...] = sa_ref[...] / ca[:, None] + sb_ref[...] / cb[:, None]


def _combine_means(sa, ca, sb, cb, n):
    bm = 400
    return pl.pallas_call(
        _comb2_body,
        grid=(n // bm,),
        in_specs=[
            pl.BlockSpec((bm, D), lambda i: (i, 0)),
            pl.BlockSpec((bm, 16), lambda i: (i, 0)),
            pl.BlockSpec((bm, D), lambda i: (i, 0)),
            pl.BlockSpec((bm, 16), lambda i: (i, 0)),
        ],
        out_specs=pl.BlockSpec((bm, D), lambda i: (i, 0)),
        out_shape=jax.ShapeDtypeStruct((n, D), jnp.float32),
    )(sa, ca, sb, cb)


_ZV = None


def _zero_rows(rows):
    zf = jnp.zeros((16,), jnp.float32)

    def body(i, carry):
        for j in range(8):
            rows[i, pl.ds(j * 16, 16)] = zf
        return carry

    lax.fori_loop(0, rows.shape[0], body, 0)


def _zero_vec(v):
    zf = jnp.zeros((16,), jnp.float32)

    def body(i, carry):
        v[pl.ds(i * 16, 16)] = zf
        return carry

    lax.fori_loop(0, v.shape[0] // 16, body, 0)


def _zero_acc_slice(rows, acc, base, n_rows):
    off = 0
    nb = rows.shape[0]
    while off < n_rows:
        nn = min(nb, n_rows - off)
        pltpu.sync_copy(rows.at[pl.ds(0, nn)], acc.at[pl.ds(base + off, nn)])
        off += nn


def _edge_stages(feat_hbm, src_hbm, dst_hbm, w_hbm, tbase, n_stage, ch, eb,
                 src_st, dst_st, w_st, dstl, rows, cntv, acc,
                 lo, r_range, filtered):
    ones = jnp.ones((16,), jnp.float32)

    def stage(si, carry):
        e0 = tbase + si * ch
        pltpu.sync_copy(src_hbm.at[pl.ds(e0, ch)], src_st)
        pltpu.sync_copy(dst_hbm.at[pl.ds(e0, ch)], dst_st)
        pltpu.sync_copy(w_hbm.at[pl.ds(e0, ch)], w_st)

        def blk(k, c1):
            pltpu.sync_copy(feat_hbm.at[src_st.at[pl.ds(k * eb, eb)]], rows)

            def grp(g, c2):
                base = k * eb + g * 16
                dv = dst_st[pl.ds(base, 16)]
                if filtered:
                    m = (dv >= lo) & (dv < lo + r_range)
                    dl = jnp.where(m, dv - lo, r_range)
                else:
                    dl = dv
                dstl[pl.ds(g * 16, 16)] = dl
                plsc.addupdate_scatter(cntv, [dl], ones)
                wvec = w_st[pl.ds(base, 16)]
                for j in range(16):
                    wj = jnp.full((16,), wvec[j], jnp.float32)
                    e = g * 16 + j
                    for d0 in range(8):
                        rows[e, pl.ds(d0 * 16, 16)] = (
                            rows[e, pl.ds(d0 * 16, 16)] * wj)
                return c2

            lax.fori_loop(0, eb // 16, grp, 0)
            pltpu.sync_copy(rows, acc.at[dstl], add=True)
            return c1

        lax.fori_loop(0, ch // eb, blk, 0)
        return carry

    lax.fori_loop(0, n_stage, stage, 0)


CH_WW = 2048
EB_WW = 64


def _make_ww_kernel(ep):
    chunk = ep // 16
    n_stage = chunk // CH_WW

    @functools.partial(
        pl.kernel,
        out_type=[
            jax.ShapeDtypeStruct((4, R16_WW, D), jnp.float32),
            jax.ShapeDtypeStruct((4, 16, R16_WW), jnp.float32),
        ],
        mesh=_mesh,
        compiler_params=_sc_params,
        scratch_types=[
            pltpu.VMEM((CH_WW,), jnp.int32),
            pltpu.VMEM((CH_WW,), jnp.int32),
            pltpu.VMEM((CH_WW,), jnp.float32),
            pltpu.VMEM((EB_WW,), jnp.int32),
            pltpu.VMEM((EB_WW, D), jnp.float32),
            pltpu.VMEM((R16_WW,), jnp.float32),
            pltpu.VMEM_SHARED((R16_WW, D), jnp.float32),
        ],
    )
    def ww_kernel(feat_hbm, src_hbm, dst_hbm, w_hbm, sum_hbm, cnt_hbm,
                  src_st, dst_st, w_st, dstl, rows, cntv, acc):
        c = lax.axis_index("c")
        s = lax.axis_index("s")
        tbase = s * chunk
        for p in range(2):
            slot = 2 * p + c
            lo = slot * R_WW
            _zero_rows(rows)
            _zero_vec(cntv)
            _zero_acc_slice(rows, acc, s * RT_WW, RT_WW)
            plsc.subcore_barrier()
            _edge_stages(feat_hbm, src_hbm, dst_hbm, w_hbm, tbase, n_stage,
                         CH_WW, EB_WW, src_st, dst_st, w_st, dstl, rows,
                         cntv, acc, lo, R_WW, filtered=True)
            plsc.subcore_barrier()
            pltpu.sync_copy(acc.at[pl.ds(s * RT_WW, RT_WW)],
                            sum_hbm.at[slot, pl.ds(s * RT_WW, RT_WW)])
            pltpu.sync_copy(cntv, cnt_hbm.at[slot, s])
            plsc.subcore_barrier()

    return ww_kernel


EB_P2 = 128


def _make_pair_kernel(ep, ch):
    chunk = ep // 16
    n_stage = chunk // ch

    @functools.partial(
        pl.kernel,
        out_type=[
            jax.ShapeDtypeStruct((N16_P2, D), jnp.float32),
            jax.ShapeDtypeStruct((16, N16_P2), jnp.float32),
            jax.ShapeDtypeStruct((N16_P2, D), jnp.float32),
            jax.ShapeDtypeStruct((16, N16_P2), jnp.float32),
        ],
        mesh=_mesh,
        compiler_params=_sc_params,
        scratch_types=[
            pltpu.VMEM((ch,), jnp.int32),
            pltpu.VMEM((ch,), jnp.int32),
            pltpu.VMEM((ch,), jnp.float32),
            pltpu.VMEM((EB_P2,), jnp.int32),
            pltpu.VMEM((EB_P2, D), jnp.float32),
            pltpu.VMEM((N16_P2,), jnp.float32),
            pltpu.VMEM_SHARED((N16_P2, D), jnp.float32),
        ],
    )
    def pair_kernel(featA_hbm, srcA_hbm, dstA_hbm, wA_hbm,
                    featB_hbm, srcB_hbm, dstB_hbm, wB_hbm,
                    sumA_hbm, cntA_hbm, sumB_hbm, cntB_hbm,
                    src_st, dst_st, w_st, dstl, rows, cntv, acc):
        c = lax.axis_index("c")
        s = lax.axis_index("s")
        tbase = s * chunk
        _zero_rows(rows)
        _zero_vec(cntv)
        _zero_acc_slice(rows, acc, s * RT_P2, RT_P2)
        plsc.subcore_barrier()

        @pl.when(c == 0)
        def _():
            _edge_stages(featA_hbm, srcA_hbm, dstA_hbm, wA_hbm, tbase,
                         n_stage, ch, EB_P2, src_st, dst_st, w_st, dstl,
                         rows, cntv, acc, 0, NT, filtered=False)

        @pl.when(c == 1)
        def _():
            _edge_stages(featB_hbm, srcB_hbm, dstB_hbm, wB_hbm, tbase,
                         n_stage, ch, EB_P2, src_st, dst_st, w_st, dstl,
                         rows, cntv, acc, 0, NT, filtered=False)

        plsc.subcore_barrier()

        @pl.when(c == 0)
        def _():
            pltpu.sync_copy(acc.at[pl.ds(s * RT_P2, RT_P2)],
                            sumA_hbm.at[pl.ds(s * RT_P2, RT_P2)])
            pltpu.sync_copy(cntv, cntA_hbm.at[s])

        @pl.when(c == 1)
        def _():
            pltpu.sync_copy(acc.at[pl.ds(s * RT_P2, RT_P2)],
                            sumB_hbm.at[pl.ds(s * RT_P2, RT_P2)])
            pltpu.sync_copy(cntv, cntB_hbm.at[s])

    return pair_kernel


def kernel(word_h, topic_h, ww_src, ww_dst, ww_w, wt_src, wt_dst, wt_w,
           wd_src, wd_dst, wd_w, td_src, td_dst, td_w, tt_src, tt_dst, tt_w,
           W_ww, b_ww, W_wt, b_wt, W_wd, b_wd, W_td, b_td, W_tt, b_tt):
    A, aG, bc, B, bB = _combine_weights(
        W_ww, b_ww, W_wt, b_wt, W_wd, b_wd, W_td, b_td, W_tt, b_tt)

    G = _matmul(word_h, A, aG)
    T2 = _matmul(topic_h, B, bB)

    sw, dw, vw, ep_ww = _pad_edges(ww_src, ww_dst, ww_w, NW, CH_WW)
    sum4, cnt4 = _make_ww_kernel(ep_ww)(G, sw, dw, vw)
    word3h = _combine_ww(sum4, cnt4, bc)
    word3 = word3h[:NW]

    ch_a, ch_b = 1024, 512
    swt, dwt, vwt, ep_a = _pad_edges(wt_src, wt_dst, wt_w, NT, ch_a)
    swd, dwd, vwd, ep_a2 = _pad_edges(wd_src, wd_dst, wd_w, ND, ch_a)
    assert ep_a == ep_a2
    s_wt, c_wt, s_wd, c_wd = _make_pair_kernel(ep_a, ch_a)(
        word3h, swt, dwt, vwt, word3h, swd, dwd, vwd)

    stt, dtt, vtt, ep_b = _pad_edges(tt_src, tt_dst, tt_w, NT, ch_b)
    std, dtd, vtd, ep_b2 = _pad_edges(td_src, td_dst, td_w, ND, ch_b)
    assert ep_b == ep_b2
    s_tt, c_tt, s_td, c_td = _make_pair_kernel(ep_b, ch_b)(
        T2, stt, dtt, vtt, T2, std, dtd, vtd)

    topic_out = _combine_means(s_wt[:NT], c_wt.T[:NT], s_tt[:NT], c_tt.T[:NT], NT)
    doc_out = _combine_means(s_wd[:ND], c_wd.T[:ND], s_td[:ND], c_td.T[:ND], ND)
    return word3, topic_out, doc_out
